# Initial kernel scaffold; baseline (speedup 1.0000x reference)
import jax
import jax.numpy as jnp
from jax.experimental import pallas as pl


def kernel(inputs, camera, img_feat_0, img_feat_1, img_feat_2):
    raise NotImplementedError("write your pallas kernel here")


# speedup vs baseline: 12.6131x; 12.6131x over previous
import functools

import jax
import jax.numpy as jnp
import numpy as np
from jax import lax
from jax.experimental import pallas as pl
from jax.experimental.pallas import tpu as pltpu
from jax.experimental.pallas import tpu_sc as plsc

P = 106038
NC, NS, L = 2, 16, 16
NW = NC * NS
B = 32
NCHUNK = 104
PPW = B * NCHUNK
P_PAD = PPW * NW
RES = (224, 112, 56)
CS = (16, 32, 64)
SCALE = (1.0, 2.0, 4.0)
NOUT = 339
FMAX = float(np.finfo(np.float32).max)


def _normal(v):
    return v / jnp.linalg.norm(v)


def _camera_mat(param):
    theta = param[0] * np.pi / 180.0
    camy = param[3] * jnp.sin(param[1] * np.pi / 180.0)
    lens = param[3] * jnp.cos(param[1] * np.pi / 180.0)
    camx = lens * jnp.cos(theta)
    camz = lens * jnp.sin(theta)
    Z = jnp.stack([camx, camy, camz])
    x = camy * jnp.cos(theta + np.pi)
    z = camy * jnp.sin(theta + np.pi)
    Y = jnp.stack([x, lens, z])
    X = jnp.cross(Y, Z)
    return jnp.stack([_normal(X), _normal(Y), _normal(Z)]), Z


def _sqrt16(v):
    i = plsc.bitcast(v, jnp.int32)
    i = jnp.int32(0x5F3759DF) - lax.shift_right_logical(i, 1)
    y = plsc.bitcast(i, jnp.float32)
    half = jnp.float32(0.5) * v
    for _ in range(3):
        y = y * (jnp.float32(1.5) - half * y * y)
    return v * y


def _recip(den):
    r = jnp.float32(1.0) / den
    r2 = r * (jnp.float32(2.0) - den * r)
    r2 = jnp.where(den == jnp.float32(0.0), r, r2)
    r3 = r2 * (jnp.float32(2.0) - den * r2)
    return jnp.where(den == jnp.float32(0.0), r, r3)


def _div_with(num, den, rr):
    q = num * rr
    return jnp.where(den == jnp.float32(0.0), q,
                     q + rr * (num - den * q))


def _nan_clean(h):
    h = jnp.where(h != h, jnp.float32(0.0), h)
    return jnp.maximum(jnp.minimum(h, jnp.float32(FMAX)), jnp.float32(-FMAX))


def _sc_body(xs_h, ys_h, zs_h, pc_h, t0, t1, t2, out_h,
             cx, cy, cz, pcv,
             i00, i01, i02, i10, i11, i12, i20, i21, i22,
             w_ref,
             b00, b01, b02, b10, b11, b12, b20, b21, b22,
             out_tile, sem):
    idx_refs = ((i00, i01, i02), (i10, i11, i12), (i20, i21, i22))
    bufs = ((b00, b01, b02), (b10, b11, b12), (b20, b21, b22))
    tbls = (t0, t1, t2)

    wid = lax.axis_index("s") * NC + lax.axis_index("c")
    base_pt = wid * PPW
    pltpu.sync_copy(xs_h.at[pl.ds(base_pt, PPW)], cx)
    pltpu.sync_copy(ys_h.at[pl.ds(base_pt, PPW)], cy)
    pltpu.sync_copy(zs_h.at[pl.ds(base_pt, PPW)], cz)
    for i in range(3):
        for j in range(3):
            pltpu.sync_copy(pc_h.at[i, j, pl.ds(base_pt, PPW)],
                            pcv.at[i * 3 + j])

    iota = lax.iota(jnp.int32, L)
    out_pvec = iota * NOUT

    def chunk_body(n, carry):
        for g in range(2):
            start = n * B + g * L
            for i in range(3):
                Xc = pcv[i * 3 + 0, pl.ds(start, L)]
                Yc = pcv[i * 3 + 1, pl.ds(start, L)]
                Zc = pcv[i * 3 + 2, pl.ds(start, L)]
                nz = -Zc
                rr = _recip(nz)
                h = jnp.float32(248.0) * _div_with(-Yc, nz, rr) + jnp.float32(112.0)
                w = jnp.float32(248.0) * _div_with(Xc, nz, rr) + jnp.float32(112.0)
                h = _nan_clean(h)
                w = _nan_clean(w)
                for s in range(3):
                    res = RES[s]
                    hi = jnp.float32(res - 1)
                    inv_sc = jnp.float32(1.0 / SCALE[s])
                    xx = jnp.minimum(jnp.maximum(h * inv_sc,
                                                 jnp.float32(0.0)), hi)
                    yy = jnp.minimum(jnp.maximum(w * inv_sc,
                                                 jnp.float32(0.0)), hi)
                    x1i = xx.astype(jnp.int32)
                    x1f = x1i.astype(jnp.float32)
                    gx = xx > x1f
                    x2i = x1i + jnp.where(gx, 1, 0).astype(jnp.int32)
                    x2f = x1f + jnp.where(gx, jnp.float32(1.0), jnp.float32(0.0))
                    y1i = yy.astype(jnp.int32)
                    y1f = y1i.astype(jnp.float32)
                    gy = yy > y1f
                    y2i = y1i + jnp.where(gy, 1, 0).astype(jnp.int32)
                    y2f = y1f + jnp.where(gy, jnp.float32(1.0), jnp.float32(0.0))
                    wa = x2f - xx
                    wb = xx - x1f
                    wc = y2f - yy
                    wd = yy - y1f
                    w_ref[s * 3 + i, 0, pl.ds(g * L, L)] = wa * wc
                    w_ref[s * 3 + i, 1, pl.ds(g * L, L)] = wb * wc
                    w_ref[s * 3 + i, 2, pl.ds(g * L, L)] = wa * wd
                    w_ref[s * 3 + i, 3, pl.ds(g * L, L)] = wb * wd
                    vbase = i * res * res
                    r1 = x1i * res + vbase
                    r2 = x2i * res + vbase
                    ir = idx_refs[s][i]
                    ir[pl.ds(0 * B + g * L, L)] = r1 + y1i
                    ir[pl.ds(1 * B + g * L, L)] = r2 + y1i
                    ir[pl.ds(2 * B + g * L, L)] = r1 + y2i
                    ir[pl.ds(3 * B + g * L, L)] = r2 + y2i

        descs = []
        for s in range(3):
            for i in range(3):
                descs.append(
                    pltpu.async_copy(tbls[s].at[idx_refs[s][i]], bufs[s][i],
                                     sem))
        for d in descs:
            d.wait()

        for g in range(2):
            start = n * B + g * L
            opv = out_pvec + g * L * NOUT
            plsc.store_scatter(out_tile, [opv], cx[pl.ds(start, L)])
            plsc.store_scatter(out_tile, [opv + 1], cy[pl.ds(start, L)])
            plsc.store_scatter(out_tile, [opv + 2], cz[pl.ds(start, L)])
            for s in range(3):
                C = CS[s]
                coff = (0, 16, 48)[s]
                rowv = [iota + (k * B + g * L) for k in range(4)]
                wv = [[w_ref[s * 3 + i, k, pl.ds(g * L, L)] for k in range(4)]
                      for i in range(3)]

                def cbody(c, carry2, s=s, g=g, rowv=rowv, wv=wv, coff=coff,
                          opv=opv):
                    colv = jnp.zeros((L,), jnp.int32) + c
                    f = []
                    for i in range(3):
                        q = [plsc.load_gather(bufs[s][i], [rowv[k], colv])
                             for k in range(4)]
                        f.append(((wv[i][0] * q[0] + wv[i][1] * q[1])
                                  + wv[i][2] * q[2]) + wv[i][3] * q[3])
                    fmax = jnp.maximum(jnp.maximum(f[0], f[1]), f[2])
                    third = jnp.float32(1.0 / 3.0)
                    m = (((f[0] + f[1]) + f[2])) * third
                    d0 = f[0] - m
                    d1 = f[1] - m
                    d2 = f[2] - m
                    var = ((d0 * d0 + d1 * d1) + d2 * d2) * third
                    st = _sqrt16(var)
                    gc = coff + c
                    plsc.store_scatter(out_tile, [opv + (3 + gc)], fmax)
                    plsc.store_scatter(out_tile, [opv + (115 + gc)], m)
                    plsc.store_scatter(out_tile, [opv + (227 + gc)], st)
                    return carry2

                lax.fori_loop(0, C, cbody, 0)

        pltpu.sync_copy(out_tile,
                        out_h.at[pl.ds((base_pt + n * B) * NOUT, B * NOUT)])
        return carry

    lax.fori_loop(0, NCHUNK, chunk_body, 0)


def kernel(inputs, camera, img_feat_0, img_feat_1, img_feat_2):
    c0, o0 = _camera_mat(camera[0])
    po = inputs @ jnp.linalg.inv(c0.T) + o0[None, :]
    pad = P_PAD - P
    pcs = []
    for i in range(3):
        ci, oi = _camera_mat(camera[i])
        pci = (po - oi[None, :]) @ ci.T
        pcs.append(jnp.pad(pci, ((0, pad), (0, 0))).T)
    pc_all = jnp.stack(pcs).astype(jnp.float32)

    xs = jnp.pad(inputs[:, 0], (0, pad))
    ys = jnp.pad(inputs[:, 1], (0, pad))
    zs = jnp.pad(inputs[:, 2], (0, pad))
    t0 = img_feat_0.reshape(3 * 224 * 224, 16)
    t1 = img_feat_1.reshape(3 * 112 * 112, 32)
    t2 = img_feat_2.reshape(3 * 56 * 56, 64)

    mesh = plsc.VectorSubcoreMesh(core_axis_name="c", subcore_axis_name="s")
    scratch = [
        pltpu.VMEM((PPW,), jnp.float32),
        pltpu.VMEM((PPW,), jnp.float32),
        pltpu.VMEM((PPW,), jnp.float32),
        pltpu.VMEM((9, PPW), jnp.float32),
    ]
    for s in range(3):
        for _ in range(3):
            scratch.append(pltpu.VMEM((4 * B,), jnp.int32))
    scratch.append(pltpu.VMEM((9, 4, B), jnp.float32))
    for s in range(3):
        for _ in range(3):
            scratch.append(pltpu.VMEM((4 * B, CS[s]), jnp.float32))
    scratch.append(pltpu.VMEM((B * NOUT,), jnp.float32))
    scratch.append(pltpu.SemaphoreType.DMA)

    run = pl.kernel(
        _sc_body,
        mesh=mesh,
        out_type=jax.ShapeDtypeStruct((P_PAD * NOUT,), jnp.float32),
        scratch_types=scratch,
        compiler_params=pltpu.CompilerParams(
            needs_layout_passes=False, use_tc_tiling_on_sc=False),
    )
    out_flat = run(xs, ys, zs, pc_all, t0, t1, t2)
    return out_flat.reshape(P_PAD, NOUT)[:P]


# speedup vs baseline: 12.6139x; 1.0001x over previous
import jax
import jax.numpy as jnp
import numpy as np
from jax import lax
from jax.experimental import pallas as pl
from jax.experimental.pallas import tpu as pltpu
from jax.experimental.pallas import tpu_sc as plsc

P = 106038
NC, NS, L = 2, 16, 16
NW = NC * NS
B = 64
G = B // L
NCHUNK = 52
PPW = B * NCHUNK
P_PAD = PPW * NW
RES = (224, 112, 56)
CS = (16, 32, 64)
INVSC = (1.0, 0.5, 0.25)
NOUT = 339
FMAX = float(np.finfo(np.float32).max)
TAILC = (P - (NW - 1) * PPW) // B
TAIL_ROWS = P - ((NW - 1) * PPW + TAILC * B)


def _normal(v):
    return v / jnp.linalg.norm(v)


def _camera_mat(param):
    theta = param[0] * np.pi / 180.0
    camy = param[3] * jnp.sin(param[1] * np.pi / 180.0)
    lens = param[3] * jnp.cos(param[1] * np.pi / 180.0)
    camx = lens * jnp.cos(theta)
    camz = lens * jnp.sin(theta)
    Z = jnp.stack([camx, camy, camz])
    x = camy * jnp.cos(theta + np.pi)
    z = camy * jnp.sin(theta + np.pi)
    Y = jnp.stack([x, lens, z])
    X = jnp.cross(Y, Z)
    return jnp.stack([_normal(X), _normal(Y), _normal(Z)]), Z


def _recip(den):
    r = jnp.float32(1.0) / den
    r2 = r * (jnp.float32(2.0) - den * r)
    r2 = jnp.where(den == jnp.float32(0.0), r, r2)
    r3 = r2 * (jnp.float32(2.0) - den * r2)
    return jnp.where(den == jnp.float32(0.0), r, r3)


def _div_with(num, den, rr):
    q = num * rr
    return jnp.where(den == jnp.float32(0.0), q,
                     q + rr * (num - den * q))


def _sqrt16(v):
    i = plsc.bitcast(v, jnp.int32)
    i = jnp.int32(0x5F3759DF) - lax.shift_right_logical(i, 1)
    y = plsc.bitcast(i, jnp.float32)
    half = jnp.float32(0.5) * v
    for _ in range(3):
        y = y * (jnp.float32(1.5) - half * y * y)
    return v * y


def _nan_clean(h):
    h = jnp.where(h != h, jnp.float32(0.0), h)
    return jnp.maximum(jnp.minimum(h, jnp.float32(FMAX)), jnp.float32(-FMAX))


def _sc_body(pc_h, t0, t1, t2, out_h,
             pc_t, i0, i1, i2, w_ref, b0, b1, b2, out_tile, sem):
    idx_refs = (i0, i1, i2)
    bufs = (b0, b1, b2)
    tbls = (t0, t1, t2)

    wid = lax.axis_index("s") * NC + lax.axis_index("c")
    base_pt = wid * PPW

    iota = lax.iota(jnp.int32, L)
    out_pvec = iota * NOUT

    def phase1(n):
        pltpu.sync_copy(pc_h.at[wid, n], pc_t)
        for g in range(G):
            gs = g * L
            for i in range(3):
                Xc = pc_t[i * 3 + 0, pl.ds(gs, L)]
                Yc = pc_t[i * 3 + 1, pl.ds(gs, L)]
                Zc = pc_t[i * 3 + 2, pl.ds(gs, L)]
                nz = -Zc
                rr = _recip(nz)
                h = jnp.float32(248.0) * _div_with(-Yc, nz, rr) \
                    + jnp.float32(112.0)
                w = jnp.float32(248.0) * _div_with(Xc, nz, rr) \
                    + jnp.float32(112.0)
                h = _nan_clean(h)
                w = _nan_clean(w)
                for s in range(3):
                    res = RES[s]
                    hi = jnp.float32(res - 1)
                    inv_sc = jnp.float32(INVSC[s])
                    xx = jnp.minimum(jnp.maximum(h * inv_sc,
                                                 jnp.float32(0.0)), hi)
                    yy = jnp.minimum(jnp.maximum(w * inv_sc,
                                                 jnp.float32(0.0)), hi)
                    x1i = xx.astype(jnp.int32)
                    x1f = x1i.astype(jnp.float32)
                    gx = xx > x1f
                    x2i = x1i + jnp.where(gx, 1, 0).astype(jnp.int32)
                    x2f = x1f + jnp.where(gx, jnp.float32(1.0),
                                          jnp.float32(0.0))
                    y1i = yy.astype(jnp.int32)
                    y1f = y1i.astype(jnp.float32)
                    gy = yy > y1f
                    y2i = y1i + jnp.where(gy, 1, 0).astype(jnp.int32)
                    y2f = y1f + jnp.where(gy, jnp.float32(1.0),
                                          jnp.float32(0.0))
                    wa = x2f - xx
                    wb = xx - x1f
                    wc = y2f - yy
                    wd = yy - y1f
                    w_ref[s * 3 + i, 0, pl.ds(gs, L)] = wa * wc
                    w_ref[s * 3 + i, 1, pl.ds(gs, L)] = wb * wc
                    w_ref[s * 3 + i, 2, pl.ds(gs, L)] = wa * wd
                    w_ref[s * 3 + i, 3, pl.ds(gs, L)] = wb * wd
                    vbase = i * res * res
                    r1 = x1i * res + vbase
                    r2 = x2i * res + vbase
                    ir = idx_refs[s]
                    ir[pl.ds((i * 4 + 0) * B + gs, L)] = r1 + y1i
                    ir[pl.ds((i * 4 + 1) * B + gs, L)] = r2 + y1i
                    ir[pl.ds((i * 4 + 2) * B + gs, L)] = r1 + y2i
                    ir[pl.ds((i * 4 + 3) * B + gs, L)] = r2 + y2i

    def fire():
        return [pltpu.async_copy(tbls[s].at[idx_refs[s]], bufs[s], sem)
                for s in range(3)]

    def combine():
        for g in range(G):
            gs = g * L
            opv = out_pvec + gs * NOUT
            rvecs = [iota + (r * B + gs) for r in range(12)]
            plsc.store_scatter(out_tile, [opv], pc_t[9, pl.ds(gs, L)])
            plsc.store_scatter(out_tile, [opv + 1], pc_t[10, pl.ds(gs, L)])
            plsc.store_scatter(out_tile, [opv + 2], pc_t[11, pl.ds(gs, L)])
            for s in range(3):
                C = CS[s]
                coff = (0, 16, 48)[s]
                wv = [[w_ref[s * 3 + i, k, pl.ds(gs, L)] for k in range(4)]
                      for i in range(3)]

                def cbody(c, carry2, s=s, rvecs=rvecs, wv=wv, coff=coff,
                          opv=opv):
                    colv = jnp.zeros((L,), jnp.int32) + c
                    f = []
                    for i in range(3):
                        q = [plsc.load_gather(
                            bufs[s], [rvecs[i * 4 + k], colv])
                            for k in range(4)]
                        f.append(((wv[i][0] * q[0] + wv[i][1] * q[1])
                                  + wv[i][2] * q[2]) + wv[i][3] * q[3])
                    fmax = jnp.maximum(jnp.maximum(f[0], f[1]), f[2])
                    third = jnp.float32(1.0 / 3.0)
                    m = ((f[0] + f[1]) + f[2]) * third
                    d0 = f[0] - m
                    d1 = f[1] - m
                    d2 = f[2] - m
                    var = ((d0 * d0 + d1 * d1) + d2 * d2) * third
                    st = _sqrt16(var)
                    gc = coff + c
                    plsc.store_scatter(out_tile, [opv + (3 + gc)], fmax)
                    plsc.store_scatter(out_tile, [opv + (115 + gc)], m)
                    plsc.store_scatter(out_tile, [opv + (227 + gc)], st)
                    return carry2

                lax.fori_loop(0, C, cbody, 0)

    def do_chunk(n):
        phase1(n)
        descs = fire()
        for d in descs:
            d.wait()
        combine()

    def chunk_body(n, carry):
        @pl.when(base_pt + (n + 1) * B <= P)
        def _():
            do_chunk(n)
            pltpu.sync_copy(
                out_tile, out_h.at[pl.ds((base_pt + n * B) * NOUT, B * NOUT)])
        return carry

    lax.fori_loop(0, NCHUNK, chunk_body, 0)

    @pl.when(wid == NW - 1)
    def _tail():
        do_chunk(TAILC)
        pltpu.sync_copy(
            out_tile.at[pl.ds(0, TAIL_ROWS * NOUT)],
            out_h.at[pl.ds((base_pt + TAILC * B) * NOUT, TAIL_ROWS * NOUT)])


def kernel(inputs, camera, img_feat_0, img_feat_1, img_feat_2):
    c0, o0 = _camera_mat(camera[0])
    po = inputs @ jnp.linalg.inv(c0.T) + o0[None, :]
    pad = P_PAD - P
    rows = []
    for i in range(3):
        ci, oi = _camera_mat(camera[i])
        pci = (po - oi[None, :]) @ ci.T
        rows.append(jnp.pad(pci, ((0, pad), (0, 0))).T)
    rows.append(jnp.pad(inputs, ((0, pad), (0, 0))).T)
    pc_h = (jnp.concatenate(rows, 0).astype(jnp.float32)
            .reshape(12, NW, NCHUNK, B).transpose(1, 2, 0, 3))

    t0 = img_feat_0.reshape(3 * 224 * 224, 16)
    t1 = img_feat_1.reshape(3 * 112 * 112, 32)
    t2 = img_feat_2.reshape(3 * 56 * 56, 64)

    mesh = plsc.VectorSubcoreMesh(core_axis_name="c", subcore_axis_name="s")
    scratch = [
        pltpu.VMEM((12, B), jnp.float32),
        pltpu.VMEM((12 * B,), jnp.int32),
        pltpu.VMEM((12 * B,), jnp.int32),
        pltpu.VMEM((12 * B,), jnp.int32),
        pltpu.VMEM((9, 4, B), jnp.float32),
        pltpu.VMEM((12 * B, 16), jnp.float32),
        pltpu.VMEM((12 * B, 32), jnp.float32),
        pltpu.VMEM((12 * B, 64), jnp.float32),
        pltpu.VMEM((B * NOUT,), jnp.float32),
        pltpu.SemaphoreType.DMA,
    ]

    run = pl.kernel(
        _sc_body,
        mesh=mesh,
        out_type=jax.ShapeDtypeStruct((P * NOUT,), jnp.float32),
        scratch_types=scratch,
        compiler_params=pltpu.CompilerParams(
            needs_layout_passes=False, use_tc_tiling_on_sc=False),
    )
    return run(pc_h, t0, t1, t2).reshape(P, NOUT)


# speedup vs baseline: 14.4608x; 1.1464x over previous
import jax
import jax.numpy as jnp
import numpy as np
from jax import lax
from jax.experimental import pallas as pl
from jax.experimental.pallas import tpu as pltpu
from jax.experimental.pallas import tpu_sc as plsc

P = 106038
NC, NS, L = 2, 16, 16
NW = NC * NS
B = 64
G = B // L
NCHUNK = 52
PPW = B * NCHUNK
P_PAD = PPW * NW
RES = (224, 112, 56)
CS = (16, 32, 64)
INVSC = (1.0, 0.5, 0.25)
NOUT = 339
FMAX = float(np.finfo(np.float32).max)
TAILC = (P - (NW - 1) * PPW) // B
TAIL_ROWS = P - ((NW - 1) * PPW + TAILC * B)


def _normal(v):
    return v / jnp.linalg.norm(v)


def _camera_mat(param):
    theta = param[0] * np.pi / 180.0
    camy = param[3] * jnp.sin(param[1] * np.pi / 180.0)
    lens = param[3] * jnp.cos(param[1] * np.pi / 180.0)
    camx = lens * jnp.cos(theta)
    camz = lens * jnp.sin(theta)
    Z = jnp.stack([camx, camy, camz])
    x = camy * jnp.cos(theta + np.pi)
    z = camy * jnp.sin(theta + np.pi)
    Y = jnp.stack([x, lens, z])
    X = jnp.cross(Y, Z)
    return jnp.stack([_normal(X), _normal(Y), _normal(Z)]), Z


def _recip(den):
    r = jnp.float32(1.0) / den
    r2 = r * (jnp.float32(2.0) - den * r)
    r2 = jnp.where(den == jnp.float32(0.0), r, r2)
    r3 = r2 * (jnp.float32(2.0) - den * r2)
    return jnp.where(den == jnp.float32(0.0), r, r3)


def _div_with(num, den, rr):
    q = num * rr
    return jnp.where(den == jnp.float32(0.0), q,
                     q + rr * (num - den * q))


def _sqrt16(v):
    i = plsc.bitcast(v, jnp.int32)
    i = jnp.int32(0x5F3759DF) - lax.shift_right_logical(i, 1)
    y = plsc.bitcast(i, jnp.float32)
    half = jnp.float32(0.5) * v
    for _ in range(3):
        y = y * (jnp.float32(1.5) - half * y * y)
    return v * y


def _nan_clean(h):
    h = jnp.where(h != h, jnp.float32(0.0), h)
    return jnp.maximum(jnp.minimum(h, jnp.float32(FMAX)), jnp.float32(-FMAX))


def _sc_body(pc_h, t0, t1, t2, out_h,
             pc_t, i0, i1, i2, w_ref, b0, b1, b2, out_tile, sem):
    idx_refs = (i0, i1, i2)
    bufs = (b0, b1, b2)
    tbls = (t0, t1, t2)

    wid = lax.axis_index("s") * NC + lax.axis_index("c")
    base_pt = wid * PPW

    iota = lax.iota(jnp.int32, L)
    out_pvec = iota * NOUT

    def phase1(n):
        pltpu.sync_copy(pc_h.at[wid, n], pc_t)
        for g in range(G):
            gs = g * L
            for i in range(3):
                Xc = pc_t[i * 3 + 0, pl.ds(gs, L)]
                Yc = pc_t[i * 3 + 1, pl.ds(gs, L)]
                Zc = pc_t[i * 3 + 2, pl.ds(gs, L)]
                nz = -Zc
                rr = _recip(nz)
                h = jnp.float32(248.0) * _div_with(-Yc, nz, rr) \
                    + jnp.float32(112.0)
                w = jnp.float32(248.0) * _div_with(Xc, nz, rr) \
                    + jnp.float32(112.0)
                h = _nan_clean(h)
                w = _nan_clean(w)
                for s in range(3):
                    res = RES[s]
                    hi = jnp.float32(res - 1)
                    inv_sc = jnp.float32(INVSC[s])
                    xx = jnp.minimum(jnp.maximum(h * inv_sc,
                                                 jnp.float32(0.0)), hi)
                    yy = jnp.minimum(jnp.maximum(w * inv_sc,
                                                 jnp.float32(0.0)), hi)
                    x1i = xx.astype(jnp.int32)
                    x1f = x1i.astype(jnp.float32)
                    gx = xx > x1f
                    x2i = x1i + jnp.where(gx, 1, 0).astype(jnp.int32)
                    x2f = x1f + jnp.where(gx, jnp.float32(1.0),
                                          jnp.float32(0.0))
                    y1i = yy.astype(jnp.int32)
                    y1f = y1i.astype(jnp.float32)
                    gy = yy > y1f
                    y2i = y1i + jnp.where(gy, 1, 0).astype(jnp.int32)
                    y2f = y1f + jnp.where(gy, jnp.float32(1.0),
                                          jnp.float32(0.0))
                    wa = x2f - xx
                    wb = xx - x1f
                    wc = y2f - yy
                    wd = yy - y1f
                    w_ref[s * 3 + i, 0, pl.ds(gs, L)] = wa * wc
                    w_ref[s * 3 + i, 1, pl.ds(gs, L)] = wb * wc
                    w_ref[s * 3 + i, 2, pl.ds(gs, L)] = wa * wd
                    w_ref[s * 3 + i, 3, pl.ds(gs, L)] = wb * wd
                    vbase = i * res * res
                    r1 = x1i * res + vbase
                    r2 = x2i * res + vbase
                    ir = idx_refs[s]
                    ir[pl.ds((i * 4 + 0) * B + gs, L)] = r1 + y1i
                    ir[pl.ds((i * 4 + 1) * B + gs, L)] = r2 + y1i
                    ir[pl.ds((i * 4 + 2) * B + gs, L)] = r1 + y2i
                    ir[pl.ds((i * 4 + 3) * B + gs, L)] = r2 + y2i

    def fire():
        return [pltpu.async_copy(tbls[s].at[idx_refs[s]], bufs[s], sem)
                for s in range(3)]

    def combine():
        for g in range(G):
            gs = g * L
            opv = out_pvec + gs * NOUT
            plsc.store_scatter(out_tile, [opv], pc_t[9, pl.ds(gs, L)])
            plsc.store_scatter(out_tile, [opv + 1], pc_t[10, pl.ds(gs, L)])
            plsc.store_scatter(out_tile, [opv + 2], pc_t[11, pl.ds(gs, L)])
            for s in range(3):
                C = CS[s]
                JJ = C // L
                coff = (0, 16, 48)[s]
                wv = [[w_ref[s * 3 + i, k, pl.ds(gs, L)] for k in range(4)]
                      for i in range(3)]

                def pbody(p0, carry2, s=s, gs=gs, wv=wv, coff=coff):
                    row0 = gs + p0
                    obase = row0 * NOUT
                    splat = jnp.zeros((L,), jnp.int32) + p0
                    f = [[None] * JJ for _ in range(3)]
                    for i in range(3):
                        ws = [wv[i][k].at[splat]
                              .get(mode="promise_in_bounds")
                              for k in range(4)]
                        for j in range(JJ):
                            q = [bufs[s][(i * 4 + k) * B + row0,
                                         pl.ds(j * L, L)]
                                 for k in range(4)]
                            f[i][j] = ((ws[0] * q[0] + ws[1] * q[1])
                                       + ws[2] * q[2]) + ws[3] * q[3]
                    third = jnp.float32(1.0 / 3.0)
                    for j in range(JJ):
                        f0, f1, f2 = f[0][j], f[1][j], f[2][j]
                        fmax = jnp.maximum(jnp.maximum(f0, f1), f2)
                        m = ((f0 + f1) + f2) * third
                        d0 = f0 - m
                        d1 = f1 - m
                        d2 = f2 - m
                        var = ((d0 * d0 + d1 * d1) + d2 * d2) * third
                        st = _sqrt16(var)
                        col = coff + j * L
                        plsc.store_scatter(out_tile,
                                           [iota + (obase + 3 + col)], fmax)
                        plsc.store_scatter(out_tile,
                                           [iota + (obase + 115 + col)], m)
                        plsc.store_scatter(out_tile,
                                           [iota + (obase + 227 + col)], st)
                    return carry2

                lax.fori_loop(0, L, pbody, 0)

    def do_chunk(n):
        phase1(n)
        descs = fire()
        for d in descs:
            d.wait()
        combine()

    def chunk_body(n, carry):
        @pl.when(base_pt + (n + 1) * B <= P)
        def _():
            do_chunk(n)
            pltpu.sync_copy(
                out_tile, out_h.at[pl.ds((base_pt + n * B) * NOUT, B * NOUT)])
        return carry

    lax.fori_loop(0, NCHUNK, chunk_body, 0)

    @pl.when(wid == NW - 1)
    def _tail():
        do_chunk(TAILC)
        pltpu.sync_copy(
            out_tile.at[pl.ds(0, TAIL_ROWS * NOUT)],
            out_h.at[pl.ds((base_pt + TAILC * B) * NOUT, TAIL_ROWS * NOUT)])


def kernel(inputs, camera, img_feat_0, img_feat_1, img_feat_2):
    c0, o0 = _camera_mat(camera[0])
    po = inputs @ jnp.linalg.inv(c0.T) + o0[None, :]
    pad = P_PAD - P
    rows = []
    for i in range(3):
        ci, oi = _camera_mat(camera[i])
        pci = (po - oi[None, :]) @ ci.T
        rows.append(jnp.pad(pci, ((0, pad), (0, 0))).T)
    rows.append(jnp.pad(inputs, ((0, pad), (0, 0))).T)
    pc_h = (jnp.concatenate(rows, 0).astype(jnp.float32)
            .reshape(12, NW, NCHUNK, B).transpose(1, 2, 0, 3))

    t0 = img_feat_0.reshape(3 * 224 * 224, 16)
    t1 = img_feat_1.reshape(3 * 112 * 112, 32)
    t2 = img_feat_2.reshape(3 * 56 * 56, 64)

    mesh = plsc.VectorSubcoreMesh(core_axis_name="c", subcore_axis_name="s")
    scratch = [
        pltpu.VMEM((12, B), jnp.float32),
        pltpu.VMEM((12 * B,), jnp.int32),
        pltpu.VMEM((12 * B,), jnp.int32),
        pltpu.VMEM((12 * B,), jnp.int32),
        pltpu.VMEM((9, 4, B), jnp.float32),
        pltpu.VMEM((12 * B, 16), jnp.float32),
        pltpu.VMEM((12 * B, 32), jnp.float32),
        pltpu.VMEM((12 * B, 64), jnp.float32),
        pltpu.VMEM((B * NOUT,), jnp.float32),
        pltpu.SemaphoreType.DMA,
    ]

    run = pl.kernel(
        _sc_body,
        mesh=mesh,
        out_type=jax.ShapeDtypeStruct((P * NOUT,), jnp.float32),
        scratch_types=scratch,
        compiler_params=pltpu.CompilerParams(
            needs_layout_passes=False, use_tc_tiling_on_sc=False),
    )
    return run(pc_h, t0, t1, t2).reshape(P, NOUT)


# speedup vs baseline: 20.6755x; 1.4298x over previous
import jax
import jax.numpy as jnp
import numpy as np
from jax import lax
from jax.experimental import pallas as pl
from jax.experimental.pallas import tpu as pltpu
from jax.experimental.pallas import tpu_sc as plsc

P = 106038
NC, NS, L = 2, 16, 16
NW = NC * NS
B = 64
G = B // L
NCHUNK = 52
PPW = B * NCHUNK
P_PAD = PPW * NW
RES = (224, 112, 56)
CS = (16, 32, 64)
INVSC = (1.0, 0.5, 0.25)
NOUT = 339
FMAX = float(np.finfo(np.float32).max)
TAILC = (P - (NW - 1) * PPW) // B
TAIL_ROWS = P - ((NW - 1) * PPW + TAILC * B)


def _normal(v):
    return v / jnp.linalg.norm(v)


def _camera_mat(param):
    theta = param[0] * np.pi / 180.0
    camy = param[3] * jnp.sin(param[1] * np.pi / 180.0)
    lens = param[3] * jnp.cos(param[1] * np.pi / 180.0)
    camx = lens * jnp.cos(theta)
    camz = lens * jnp.sin(theta)
    Z = jnp.stack([camx, camy, camz])
    x = camy * jnp.cos(theta + np.pi)
    z = camy * jnp.sin(theta + np.pi)
    Y = jnp.stack([x, lens, z])
    X = jnp.cross(Y, Z)
    return jnp.stack([_normal(X), _normal(Y), _normal(Z)]), Z


def _recip(den):
    r = jnp.float32(1.0) / den
    r2 = r * (jnp.float32(2.0) - den * r)
    r2 = jnp.where(den == jnp.float32(0.0), r, r2)
    r3 = r2 * (jnp.float32(2.0) - den * r2)
    return jnp.where(den == jnp.float32(0.0), r, r3)


def _div_with(num, den, rr):
    q = num * rr
    return jnp.where(den == jnp.float32(0.0), q,
                     q + rr * (num - den * q))


def _sqrt16(v):
    i = plsc.bitcast(v, jnp.int32)
    i = jnp.int32(0x5F3759DF) - lax.shift_right_logical(i, 1)
    y = plsc.bitcast(i, jnp.float32)
    half = jnp.float32(0.5) * v
    for _ in range(3):
        y = y * (jnp.float32(1.5) - half * y * y)
    return v * y


def _nan_clean(h):
    h = jnp.where(h != h, jnp.float32(0.0), h)
    return jnp.maximum(jnp.minimum(h, jnp.float32(FMAX)), jnp.float32(-FMAX))


def _sc_body(pc_h, t0, t1, t2, out_h,
             pc_t, i0, i1, i2, w_ref, b0, b1, b2, out_tile, sem):
    idx_refs = (i0, i1, i2)
    bufs = (b0, b1, b2)
    tbls = (t0, t1, t2)

    wid = lax.axis_index("s") * NC + lax.axis_index("c")
    base_pt = wid * PPW

    iota = lax.iota(jnp.int32, L)
    out_pvec = iota * NOUT

    def phase1(n):
        pltpu.sync_copy(pc_h.at[wid, n], pc_t)
        for g in range(G):
            gs = g * L
            for i in range(3):
                Xc = pc_t[i * 3 + 0, pl.ds(gs, L)]
                Yc = pc_t[i * 3 + 1, pl.ds(gs, L)]
                Zc = pc_t[i * 3 + 2, pl.ds(gs, L)]
                nz = -Zc
                rr = _recip(nz)
                h = jnp.float32(248.0) * _div_with(-Yc, nz, rr) \
                    + jnp.float32(112.0)
                w = jnp.float32(248.0) * _div_with(Xc, nz, rr) \
                    + jnp.float32(112.0)
                h = _nan_clean(h)
                w = _nan_clean(w)
                for s in range(3):
                    res = RES[s]
                    hi = jnp.float32(res - 1)
                    inv_sc = jnp.float32(INVSC[s])
                    xx = jnp.minimum(jnp.maximum(h * inv_sc,
                                                 jnp.float32(0.0)), hi)
                    yy = jnp.minimum(jnp.maximum(w * inv_sc,
                                                 jnp.float32(0.0)), hi)
                    x1i = xx.astype(jnp.int32)
                    x1f = x1i.astype(jnp.float32)
                    gx = xx > x1f
                    x2i = x1i + jnp.where(gx, 1, 0).astype(jnp.int32)
                    x2f = x1f + jnp.where(gx, jnp.float32(1.0),
                                          jnp.float32(0.0))
                    y1i = yy.astype(jnp.int32)
                    y1f = y1i.astype(jnp.float32)
                    gy = yy > y1f
                    y2i = y1i + jnp.where(gy, 1, 0).astype(jnp.int32)
                    y2f = y1f + jnp.where(gy, jnp.float32(1.0),
                                          jnp.float32(0.0))
                    wa = x2f - xx
                    wb = xx - x1f
                    wc = y2f - yy
                    wd = yy - y1f
                    w_ref[s * 3 + i, 0, pl.ds(gs, L)] = wa * wc
                    w_ref[s * 3 + i, 1, pl.ds(gs, L)] = wb * wc
                    w_ref[s * 3 + i, 2, pl.ds(gs, L)] = wa * wd
                    w_ref[s * 3 + i, 3, pl.ds(gs, L)] = wb * wd
                    x1c = jnp.minimum(x1i, jnp.int32(res - 2))
                    y1c = jnp.minimum(y1i, jnp.int32(res - 2))
                    idx_refs[s][pl.ds(i * B + gs, L)] = \
                        (x1c * res + y1c) + i * res * res

    def fire():
        return [pltpu.async_copy(tbls[s].at[idx_refs[s]], bufs[s], sem)
                for s in range(3)]

    def combine():
        for g in range(G):
            gs = g * L
            opv = out_pvec + gs * NOUT
            plsc.store_scatter(out_tile, [opv], pc_t[9, pl.ds(gs, L)])
            plsc.store_scatter(out_tile, [opv + 1], pc_t[10, pl.ds(gs, L)])
            plsc.store_scatter(out_tile, [opv + 2], pc_t[11, pl.ds(gs, L)])
            for s in range(3):
                C = CS[s]
                JJ = C // L
                coff = (0, 16, 48)[s]
                wv = [[w_ref[s * 3 + i, k, pl.ds(gs, L)] for k in range(4)]
                      for i in range(3)]

                def pbody(p0, carry2, s=s, gs=gs, wv=wv, coff=coff):
                    row0 = gs + p0
                    obase = row0 * NOUT
                    splat = jnp.zeros((L,), jnp.int32) + p0
                    C = CS[s]
                    qoff = (0, 2 * C, C, 3 * C)
                    f = [[None] * JJ for _ in range(3)]
                    for i in range(3):
                        ws = [wv[i][k].at[splat]
                              .get(mode="promise_in_bounds")
                              for k in range(4)]
                        for j in range(JJ):
                            q = [bufs[s][i * B + row0,
                                         pl.ds(qoff[k] + j * L, L)]
                                 for k in range(4)]
                            f[i][j] = ((ws[0] * q[0] + ws[1] * q[1])
                                       + ws[2] * q[2]) + ws[3] * q[3]
                    third = jnp.float32(1.0 / 3.0)
                    for j in range(JJ):
                        f0, f1, f2 = f[0][j], f[1][j], f[2][j]
                        fmax = jnp.maximum(jnp.maximum(f0, f1), f2)
                        m = ((f0 + f1) + f2) * third
                        d0 = f0 - m
                        d1 = f1 - m
                        d2 = f2 - m
                        var = ((d0 * d0 + d1 * d1) + d2 * d2) * third
                        st = _sqrt16(var)
                        col = coff + j * L
                        plsc.store_scatter(out_tile,
                                           [iota + (obase + 3 + col)], fmax)
                        plsc.store_scatter(out_tile,
                                           [iota + (obase + 115 + col)], m)
                        plsc.store_scatter(out_tile,
                                           [iota + (obase + 227 + col)], st)
                    return carry2

                lax.fori_loop(0, L, pbody, 0)

    def do_chunk(n):
        phase1(n)
        descs = fire()
        for d in descs:
            d.wait()
        combine()

    def chunk_body(n, carry):
        @pl.when(base_pt + (n + 1) * B <= P)
        def _():
            do_chunk(n)
            pltpu.sync_copy(
                out_tile, out_h.at[pl.ds((base_pt + n * B) * NOUT, B * NOUT)])
        return carry

    lax.fori_loop(0, NCHUNK, chunk_body, 0)

    @pl.when(wid == NW - 1)
    def _tail():
        do_chunk(TAILC)
        pltpu.sync_copy(
            out_tile.at[pl.ds(0, TAIL_ROWS * NOUT)],
            out_h.at[pl.ds((base_pt + TAILC * B) * NOUT, TAIL_ROWS * NOUT)])


def kernel(inputs, camera, img_feat_0, img_feat_1, img_feat_2):
    c0, o0 = _camera_mat(camera[0])
    po = inputs @ jnp.linalg.inv(c0.T) + o0[None, :]
    pad = P_PAD - P
    rows = []
    for i in range(3):
        ci, oi = _camera_mat(camera[i])
        pci = (po - oi[None, :]) @ ci.T
        rows.append(jnp.pad(pci, ((0, pad), (0, 0))).T)
    rows.append(jnp.pad(inputs, ((0, pad), (0, 0))).T)
    pc_h = (jnp.concatenate(rows, 0).astype(jnp.float32)
            .reshape(12, NW, NCHUNK, B).transpose(1, 2, 0, 3))

    def _quad(feat, res):
        ty = jnp.concatenate([feat[:, :, 1:, :], feat[:, :, -1:, :]], 2)
        q = jnp.concatenate([feat, ty], 3)
        qx = jnp.concatenate([q[:, 1:, :, :], q[:, -1:, :, :]], 1)
        return jnp.concatenate([q, qx], 3).reshape(3 * res * res, -1)

    t0 = _quad(img_feat_0, 224)
    t1 = _quad(img_feat_1, 112)
    t2 = _quad(img_feat_2, 56)

    mesh = plsc.VectorSubcoreMesh(core_axis_name="c", subcore_axis_name="s")
    scratch = [
        pltpu.VMEM((12, B), jnp.float32),
        pltpu.VMEM((3 * B,), jnp.int32),
        pltpu.VMEM((3 * B,), jnp.int32),
        pltpu.VMEM((3 * B,), jnp.int32),
        pltpu.VMEM((9, 4, B), jnp.float32),
        pltpu.VMEM((3 * B, 64), jnp.float32),
        pltpu.VMEM((3 * B, 128), jnp.float32),
        pltpu.VMEM((3 * B, 256), jnp.float32),
        pltpu.VMEM((B * NOUT,), jnp.float32),
        pltpu.SemaphoreType.DMA,
    ]

    run = pl.kernel(
        _sc_body,
        mesh=mesh,
        out_type=jax.ShapeDtypeStruct((P * NOUT,), jnp.float32),
        scratch_types=scratch,
        compiler_params=pltpu.CompilerParams(
            needs_layout_passes=False, use_tc_tiling_on_sc=False),
    )
    return run(pc_h, t0, t1, t2).reshape(P, NOUT)
